# baseline (speedup 1.0000x reference)
import functools

import jax
import jax.numpy as jnp
from jax import lax
from jax.experimental import pallas as pl
from jax.experimental.pallas import tpu as pltpu
from jax.experimental.pallas import tpu_sc as plsc

R, S, D = 16384, 200, 32
V = 1000
L = 16
RB = R // 128
NU = S * RB
NW = 32
UPW = NU // NW

_mesh = plsc.VectorSubcoreMesh(core_axis_name="c", subcore_axis_name="s")


@functools.partial(
    pl.kernel,
    mesh=_mesh,
    out_type=jax.ShapeDtypeStruct((S, D // 8, RB, 8, 128), jnp.float32),
    scratch_types=[
        pltpu.VMEM((V * (D + 1),), jnp.float32),
        pltpu.VMEM((4, 128), jnp.int32),
        pltpu.VMEM((4, D // 8, 8, 128), jnp.float32),
        pltpu.SemaphoreType.DMA((4,)),
        pltpu.SemaphoreType.DMA((4,)),
    ],
    compiler_params=pltpu.CompilerParams(
        use_tc_tiling_on_sc=False, needs_layout_passes=False
    ),
)
def _sc_lookup(table_hbm, idx_hbm, out_hbm, table_v, idx_v, blk_v, isem, ssem):
    wid = lax.axis_index("s") * 2 + lax.axis_index("c")
    u0 = wid * UPW

    pltpu.sync_copy(table_hbm, table_v)

    def coords(u):
        s = u // 128
        rb = lax.rem(u, 128)
        return s // 8, lax.rem(s, 8), s, rb

    def fire_idx(u, b):
        sb, si, _, rb = coords(lax.min(u, NU - 1))
        pltpu.async_copy(idx_hbm.at[sb, rb, si], idx_v.at[b], isem.at[b])

    def wait_idx(b):
        pltpu.make_async_copy(
            idx_hbm.at[0, 0, 0], idx_v.at[b], isem.at[b]
        ).wait()

    def compute(b):
        @plsc.parallel_loop(0, 128 // L, unroll=2)
        def _(v):
            vs = pl.multiple_of(v * L, L)
            iv = idx_v[b, pl.ds(vs, L)]
            fm = iv * (D + 1)
            for d in range(D):
                g = plsc.load_gather(table_v, [fm + d])
                blk_v[b, d // 8, d % 8, pl.ds(vs, L)] = g

    def fire_store(u, b):
        _, _, s, rb = coords(u)
        pltpu.async_copy(blk_v.at[b], out_hbm.at[s, :, rb], ssem.at[b])

    def wait_store(b):
        pltpu.make_async_copy(
            blk_v.at[b], out_hbm.at[0, :, 0], ssem.at[b]
        ).wait()

    NB = 4
    for b in range(NB):
        fire_idx(u0 + b, b)
    for b in range(NB):
        wait_idx(b)
        compute(b)
        fire_store(u0 + b, b)
        fire_idx(u0 + b + NB, b)

    def body(g, _):
        for b in range(NB):
            u = u0 + NB + NB * g + b
            wait_idx(b)
            wait_store(b)
            compute(b)
            fire_store(u, b)
            fire_idx(u + NB, b)
        return ()

    lax.fori_loop(0, (UPW - NB) // NB, body, (), unroll=False)

    for b in range(NB):
        wait_idx(b)
        wait_store(b)


def kernel(visit_segments, embedding_weight):
    idx_t = (
        visit_segments.astype(jnp.int32)
        .reshape(128, 128, 25, 8)
        .transpose(2, 0, 3, 1)
    )
    table_pad = jnp.pad(embedding_weight, ((0, 0), (0, 1))).reshape(-1)
    out_t = _sc_lookup(table_pad, idx_t)
    return out_t.transpose(2, 4, 0, 1, 3).reshape(R, S, D)


# speedup vs baseline: 1.6767x; 1.6767x over previous
import functools

import jax
import jax.numpy as jnp
from jax import lax
from jax.experimental import pallas as pl
from jax.experimental.pallas import tpu as pltpu
from jax.experimental.pallas import tpu_sc as plsc

R, S, D = 16384, 200, 32
V = 1000
L = 16
RB = R // 128
NU = S * RB
NW = 32
UPW = NU // NW

_mesh = plsc.VectorSubcoreMesh(core_axis_name="c", subcore_axis_name="s")


@functools.partial(
    pl.kernel,
    mesh=_mesh,
    out_type=jax.ShapeDtypeStruct((S, D // 8, RB, 8, 128), jnp.float32),
    scratch_types=[
        pltpu.VMEM((V * (D + 1),), jnp.float32),
        pltpu.VMEM((4, 128), jnp.int32),
        pltpu.VMEM((4, D // 8, 8, 128), jnp.float32),
        pltpu.SemaphoreType.DMA((4,)),
        pltpu.SemaphoreType.DMA((4,)),
    ],
    compiler_params=pltpu.CompilerParams(
        use_tc_tiling_on_sc=False, needs_layout_passes=False
    ),
)
def _sc_lookup(table_hbm, idx_hbm, out_hbm, table_v, idx_v, blk_v, isem, ssem):
    wid = lax.axis_index("s") * 2 + lax.axis_index("c")
    u0 = wid * UPW

    table_cp = pltpu.make_async_copy(table_hbm, table_v, ssem.at[0])
    table_cp.start()

    def coords(u):
        s = u // 128
        rb = lax.rem(u, 128)
        return s // 8, lax.rem(s, 8), s, rb

    def fire_idx(u, b):
        sb, si, _, rb = coords(lax.min(u, NU - 1))
        pltpu.async_copy(idx_hbm.at[sb, rb, si], idx_v.at[b], isem.at[b])

    def wait_idx(b):
        pltpu.make_async_copy(
            idx_hbm.at[0, 0, 0], idx_v.at[b], isem.at[b]
        ).wait()

    def compute(b):
        @plsc.parallel_loop(0, 128 // L, unroll=4)
        def _(v):
            vs = pl.multiple_of(v * L, L)
            iv = idx_v[b, pl.ds(vs, L)]
            fm = iv * (D + 1)
            for d in range(D):
                g = plsc.load_gather(table_v, [fm + d])
                blk_v[b, d // 8, d % 8, pl.ds(vs, L)] = g

    def fire_store(u, b):
        _, _, s, rb = coords(u)
        pltpu.async_copy(blk_v.at[b], out_hbm.at[s, :, rb], ssem.at[b])

    def wait_store(b):
        pltpu.make_async_copy(
            blk_v.at[b], out_hbm.at[0, :, 0], ssem.at[b]
        ).wait()

    NB = 4
    for b in range(NB):
        fire_idx(u0 + b, b)
    table_cp.wait()
    for b in range(NB):
        wait_idx(b)
        compute(b)
        fire_store(u0 + b, b)
        fire_idx(u0 + b + NB, b)

    def body(g, _):
        for b in range(NB):
            u = u0 + NB + NB * g + b
            wait_idx(b)
            wait_store(b)
            compute(b)
            fire_store(u, b)
            fire_idx(u + NB, b)
        return ()

    lax.fori_loop(0, (UPW - NB) // NB, body, (), unroll=False)

    for b in range(NB):
        wait_idx(b)
        wait_store(b)


def kernel(visit_segments, embedding_weight):
    idx_t = (
        visit_segments.astype(jnp.int32)
        .reshape(128, 128, 25, 8)
        .transpose(2, 0, 3, 1)
    )
    table_pad = jnp.pad(embedding_weight, ((0, 0), (0, 1))).reshape(-1)
    out_t = _sc_lookup(table_pad, idx_t)
    return out_t.transpose(2, 4, 0, 1, 3).reshape(R, S, D)


# speedup vs baseline: 1.6793x; 1.0015x over previous
import functools

import jax
import jax.numpy as jnp
from jax import lax
from jax.experimental import pallas as pl
from jax.experimental.pallas import tpu as pltpu
from jax.experimental.pallas import tpu_sc as plsc

R, S, D = 16384, 200, 32
V = 1000
L = 16
RB = R // 128
NU = S * RB
NW = 32
UPW = NU // NW

_mesh = plsc.VectorSubcoreMesh(core_axis_name="c", subcore_axis_name="s")


@functools.partial(
    pl.kernel,
    mesh=_mesh,
    out_type=jax.ShapeDtypeStruct((S, D // 8, RB, 8, 128), jnp.float32),
    scratch_types=[
        pltpu.VMEM((V * (D + 3),), jnp.float32),
        pltpu.VMEM((4, 128), jnp.int32),
        pltpu.VMEM((4, D // 8, 8, 128), jnp.float32),
        pltpu.SemaphoreType.DMA((4,)),
        pltpu.SemaphoreType.DMA((4,)),
    ],
    compiler_params=pltpu.CompilerParams(
        use_tc_tiling_on_sc=False, needs_layout_passes=False
    ),
)
def _sc_lookup(table_hbm, idx_hbm, out_hbm, table_v, idx_v, blk_v, isem, ssem):
    wid = lax.axis_index("s") * 2 + lax.axis_index("c")
    u0 = wid * UPW

    table_cp = pltpu.make_async_copy(table_hbm, table_v, ssem.at[0])
    table_cp.start()

    def coords(u):
        s = u // 128
        rb = lax.rem(u, 128)
        return s // 8, lax.rem(s, 8), s, rb

    def fire_idx(u, b):
        sb, si, _, rb = coords(lax.min(u, NU - 1))
        pltpu.async_copy(idx_hbm.at[sb, rb, si], idx_v.at[b], isem.at[b])

    def wait_idx(b):
        pltpu.make_async_copy(
            idx_hbm.at[0, 0, 0], idx_v.at[b], isem.at[b]
        ).wait()

    def compute(b):
        @plsc.parallel_loop(0, 128 // L, unroll=4)
        def _(v):
            vs = pl.multiple_of(v * L, L)
            iv = idx_v[b, pl.ds(vs, L)]
            fm = iv * (D + 3)
            for d in range(D):
                g = plsc.load_gather(table_v, [fm + d])
                blk_v[b, d // 8, d % 8, pl.ds(vs, L)] = g

    def fire_store(u, b):
        _, _, s, rb = coords(u)
        pltpu.async_copy(blk_v.at[b], out_hbm.at[s, :, rb], ssem.at[b])

    def wait_store(b):
        pltpu.make_async_copy(
            blk_v.at[b], out_hbm.at[0, :, 0], ssem.at[b]
        ).wait()

    NB = 4
    for b in range(NB):
        fire_idx(u0 + b, b)
    table_cp.wait()
    for b in range(NB):
        wait_idx(b)
        compute(b)
        fire_store(u0 + b, b)
        fire_idx(u0 + b + NB, b)

    def body(g, _):
        for b in range(NB):
            u = u0 + NB + NB * g + b
            wait_idx(b)
            wait_store(b)
            compute(b)
            fire_store(u, b)
            fire_idx(u + NB, b)
        return ()

    lax.fori_loop(0, (UPW - NB) // NB, body, (), unroll=False)

    for b in range(NB):
        wait_idx(b)
        wait_store(b)


def kernel(visit_segments, embedding_weight):
    idx_t = (
        visit_segments.astype(jnp.int32)
        .reshape(128, 128, 25, 8)
        .transpose(2, 0, 3, 1)
    )
    table_pad = jnp.pad(embedding_weight, ((0, 0), (0, 3))).reshape(-1)
    out_t = _sc_lookup(table_pad, idx_t)
    return out_t.transpose(2, 4, 0, 1, 3).reshape(R, S, D)
